# baseline (speedup 1.0000x reference)
import functools

import jax
import jax.numpy as jnp
from jax import lax
from jax.experimental import pallas as pl
from jax.experimental.pallas import tpu as pltpu
from jax.experimental.pallas import tpu_sc as plsc


_SCALE = 32.0

_NC = 2
_NS = 16
_NW = _NC * _NS

_B = 4
_SEQ = 8192
_D = 1024
_ROWS_W = _SEQ // _NW
_R = 8
_NCHUNK = _ROWS_W // _R
_VPR = _D // 16
_NSET = 3


def _fma_chunk4(x4, pebuf):
    @plsc.parallel_loop(0, _R * _VPR, step=1, unroll=8)
    def body(k):
        i = k >> 6
        sl = pl.ds((k & (_VPR - 1)) * 16, 16)
        pe = pebuf[i, sl]
        for xb in x4:
            xb[i, sl] = xb[i, sl] * _SCALE + pe


def _sc_body(x_hbm, pe_hbm, out_hbm, *scratch):
    xbufs = scratch[0:_NSET * _B]
    pebufs = scratch[_NSET * _B:_NSET * _B + 2]
    lsems = scratch[_NSET * _B + 2:_NSET * _B + 2 + _NSET]
    ssems = scratch[_NSET * _B + 2 + _NSET:_NSET * _B + 2 + 2 * _NSET]
    spe = scratch[-1]

    wid = lax.axis_index("s") * _NC + lax.axis_index("c")
    row0 = wid * _ROWS_W

    def xset(c):
        s = c % _NSET
        return xbufs[_B * s:_B * (s + 1)]

    def issue_loads(c):
        bs = xset(c)
        sem = lsems[c % _NSET]
        return [
            pltpu.async_copy(
                x_hbm.at[b, pl.ds(row0 + c * _R, _R), :], bs[b], sem)
            for b in range(_B)
        ]

    def issue_stores(c):
        bs = xset(c)
        sem = ssems[c % _NSET]
        return [
            pltpu.async_copy(
                bs[b], out_hbm.at[b, pl.ds(row0 + c * _R, _R), :], sem)
            for b in range(_B)
        ]

    pltpu.sync_copy(pe_hbm.at[pl.ds(row0, _R), :], pebufs[0])

    loads = {}
    pe_loads = {}
    stores = {}
    for c in range(min(_NSET - 1, _NCHUNK)):
        loads[c] = issue_loads(c)
    for c in range(_NCHUNK):
        if c + 1 < _NCHUNK:
            pe_loads[c + 1] = pltpu.async_copy(
                pe_hbm.at[pl.ds(row0 + (c + 1) * _R, _R), :],
                pebufs[(c + 1) % 2], spe)
        if c > 0:
            pe_loads[c].wait()
        for h in loads[c]:
            h.wait()
        _fma_chunk4(xset(c), pebufs[c % 2])
        stores[c] = issue_stores(c)
        if c + _NSET - 1 < _NCHUNK:
            if c - 1 >= 0:
                for h in stores[c - 1]:
                    h.wait()
            loads[c + _NSET - 1] = issue_loads(c + _NSET - 1)
    for c in range(max(0, _NCHUNK - _NSET), _NCHUNK):
        for h in stores[c]:
            h.wait()


def _sc_call(x, pos_emb):
    mesh = plsc.VectorSubcoreMesh(core_axis_name="c", subcore_axis_name="s")
    run = functools.partial(
        pl.kernel,
        mesh=mesh,
        out_type=jax.ShapeDtypeStruct((_B, _SEQ, _D), jnp.float32),
        scratch_types=(
            [pltpu.VMEM((_R, _D), jnp.float32)] * (_NSET * _B + 2)
            + [pltpu.SemaphoreType.DMA] * (2 * _NSET + 1)
        ),
    )(_sc_body)
    return run(x, pos_emb)


def kernel(x, pos_emb):
    b, s, d = x.shape
    return _sc_call(x, pos_emb[:s])


# speedup vs baseline: 1.0082x; 1.0082x over previous
import functools

import jax
import jax.numpy as jnp
from jax import lax
from jax.experimental import pallas as pl
from jax.experimental.pallas import tpu as pltpu
from jax.experimental.pallas import tpu_sc as plsc


_SCALE = 32.0

_NC = 2
_NS = 16
_NW = _NC * _NS

_B = 4
_SEQ = 8192
_D = 1024
_ROWS_W = _SEQ // _NW
_R = 8
_NCHUNK = _ROWS_W // _R
_VPR = _D // 16
_NSET = 3


def _fma_chunk4(x4, pebuf):
    @plsc.parallel_loop(0, _R * _VPR, step=1, unroll=4)
    def body(k):
        i = k >> 6
        sl = pl.ds((k & (_VPR - 1)) * 16, 16)
        pe = pebuf[i, sl]
        for xb in x4:
            xb[i, sl] = xb[i, sl] * _SCALE + pe


def _sc_body(x_hbm, pe_hbm, out_hbm, *scratch):
    xbufs = scratch[0:_NSET * _B]
    pebufs = scratch[_NSET * _B:_NSET * _B + 2]
    lsems = scratch[_NSET * _B + 2:_NSET * _B + 2 + _NSET]
    ssems = scratch[_NSET * _B + 2 + _NSET:_NSET * _B + 2 + 2 * _NSET]
    spe = scratch[-1]

    wid = lax.axis_index("s") * _NC + lax.axis_index("c")
    row0 = wid * _ROWS_W

    def xset(c):
        s = c % _NSET
        return xbufs[_B * s:_B * (s + 1)]

    def issue_loads(c):
        bs = xset(c)
        sem = lsems[c % _NSET]
        return [
            pltpu.async_copy(
                x_hbm.at[b, pl.ds(row0 + c * _R, _R), :], bs[b], sem)
            for b in range(_B)
        ]

    def issue_stores(c):
        bs = xset(c)
        sem = ssems[c % _NSET]
        return [
            pltpu.async_copy(
                bs[b], out_hbm.at[b, pl.ds(row0 + c * _R, _R), :], sem)
            for b in range(_B)
        ]

    pltpu.sync_copy(pe_hbm.at[pl.ds(row0, _R), :], pebufs[0])

    loads = {}
    pe_loads = {}
    stores = {}
    for c in range(min(_NSET - 1, _NCHUNK)):
        loads[c] = issue_loads(c)
    for c in range(_NCHUNK):
        if c + 1 < _NCHUNK:
            pe_loads[c + 1] = pltpu.async_copy(
                pe_hbm.at[pl.ds(row0 + (c + 1) * _R, _R), :],
                pebufs[(c + 1) % 2], spe)
        if c > 0:
            pe_loads[c].wait()
        for h in loads[c]:
            h.wait()
        _fma_chunk4(xset(c), pebufs[c % 2])
        stores[c] = issue_stores(c)
        if c + _NSET - 1 < _NCHUNK:
            if c - 1 >= 0:
                for h in stores[c - 1]:
                    h.wait()
            loads[c + _NSET - 1] = issue_loads(c + _NSET - 1)
    for c in range(max(0, _NCHUNK - _NSET), _NCHUNK):
        for h in stores[c]:
            h.wait()


def _sc_call(x, pos_emb):
    mesh = plsc.VectorSubcoreMesh(core_axis_name="c", subcore_axis_name="s")
    run = functools.partial(
        pl.kernel,
        mesh=mesh,
        out_type=jax.ShapeDtypeStruct((_B, _SEQ, _D), jnp.float32),
        scratch_types=(
            [pltpu.VMEM((_R, _D), jnp.float32)] * (_NSET * _B + 2)
            + [pltpu.SemaphoreType.DMA] * (2 * _NSET + 1)
        ),
    )(_sc_body)
    return run(x, pos_emb)


def kernel(x, pos_emb):
    b, s, d = x.shape
    assert (b, s, d) == (_B, _SEQ, _D), (b, s, d)
    return _sc_call(x, pos_emb[:s])


# speedup vs baseline: 1.0563x; 1.0478x over previous
import functools

import jax
import jax.numpy as jnp
from jax import lax
from jax.experimental import pallas as pl
from jax.experimental.pallas import tpu as pltpu
from jax.experimental.pallas import tpu_sc as plsc


_SCALE = 32.0

_NC = 2
_NS = 16
_NW = _NC * _NS

_B = 4
_SEQ = 8192
_D = 1024
_ROWS_W = _SEQ // _NW
_R = 4
_NCHUNK = _ROWS_W // _R
_VPR = _D // 16
_NSET = 4


def _fma_chunk4(x4, pebuf):
    @plsc.parallel_loop(0, _R * _VPR, step=1, unroll=4)
    def body(k):
        i = k >> 6
        sl = pl.ds((k & (_VPR - 1)) * 16, 16)
        pe = pebuf[i, sl]
        for xb in x4:
            xb[i, sl] = xb[i, sl] * _SCALE + pe


def _sc_body(x_hbm, pe_hbm, out_hbm, *scratch):
    xbufs = scratch[0:_NSET * _B]
    pebufs = scratch[_NSET * _B:_NSET * _B + 2]
    lsems = scratch[_NSET * _B + 2:_NSET * _B + 2 + _NSET]
    ssems = scratch[_NSET * _B + 2 + _NSET:_NSET * _B + 2 + 2 * _NSET]
    spe = scratch[-1]

    wid = lax.axis_index("s") * _NC + lax.axis_index("c")
    row0 = wid * _ROWS_W

    def xset(k):
        return xbufs[_B * k:_B * (k + 1)]

    def issue_loads(c, k):
        for b in range(_B):
            pltpu.async_copy(
                x_hbm.at[b, pl.ds(row0 + c * _R, _R), :],
                xset(k)[b], lsems[k])

    def wait_loads(k):
        for b in range(_B):
            pltpu.make_async_copy(
                x_hbm.at[b, pl.ds(row0, _R), :], xset(k)[b],
                lsems[k]).wait()

    def issue_stores(c, k):
        for b in range(_B):
            pltpu.async_copy(
                xset(k)[b], out_hbm.at[b, pl.ds(row0 + c * _R, _R), :],
                ssems[k])

    def wait_stores(k):
        for b in range(_B):
            pltpu.make_async_copy(
                xset(k)[b], out_hbm.at[b, pl.ds(row0, _R), :],
                ssems[k]).wait()

    def issue_pe_kb(c, kb):
        pltpu.async_copy(
            pe_hbm.at[pl.ds(row0 + c * _R, _R), :], pebufs[kb], spe)

    def wait_pe(kb):
        pltpu.make_async_copy(
            pe_hbm.at[pl.ds(row0, _R), :], pebufs[kb], spe).wait()

    def slot(c, k, kb, *, pe_wait, pe_next, st_wait, ld_next):
        if pe_next:
            issue_pe_kb(c + 1, 1 - kb)
        if pe_wait:
            wait_pe(kb)
        wait_loads(k)
        _fma_chunk4(xset(k), pebufs[kb])
        issue_stores(c, k)
        if st_wait:
            wait_stores((k + 2) % _NSET)
        if ld_next:
            issue_loads(c + 2, (k + 2) % _NSET)

    pltpu.sync_copy(pe_hbm.at[pl.ds(row0, _R), :], pebufs[0])
    issue_loads(0, 0)
    issue_loads(1, 1)
    slot(0, 0, 0, pe_wait=False, pe_next=True, st_wait=False, ld_next=True)
    slot(1, 1, 1, pe_wait=True, pe_next=True, st_wait=False, ld_next=True)

    n_groups = (_NCHUNK - 4) // _NSET

    def group(g, carry):
        c0 = 2 + g * _NSET
        for i in range(_NSET):
            slot(c0 + i, (2 + i) % _NSET, i % 2,
                 pe_wait=True, pe_next=True, st_wait=True, ld_next=True)
        return carry
    lax.fori_loop(0, n_groups, group, 0)

    slot(_NCHUNK - 2, (_NCHUNK - 2) % _NSET, (_NCHUNK - 2) % 2,
         pe_wait=True, pe_next=True, st_wait=True, ld_next=False)
    slot(_NCHUNK - 1, (_NCHUNK - 1) % _NSET, (_NCHUNK - 1) % 2,
         pe_wait=True, pe_next=False, st_wait=False, ld_next=False)
    for c in range(_NCHUNK - 3, _NCHUNK):
        wait_stores(c % _NSET)


def _sc_call(x, pos_emb):
    mesh = plsc.VectorSubcoreMesh(core_axis_name="c", subcore_axis_name="s")
    run = functools.partial(
        pl.kernel,
        mesh=mesh,
        out_type=jax.ShapeDtypeStruct((_B, _SEQ, _D), jnp.float32),
        scratch_types=(
            [pltpu.VMEM((_R, _D), jnp.float32)] * (_NSET * _B + 2)
            + [pltpu.SemaphoreType.DMA] * (2 * _NSET + 1)
        ),
    )(_sc_body)
    return run(x, pos_emb)


def kernel(x, pos_emb):
    b, s, d = x.shape
    assert (b, s, d) == (_B, _SEQ, _D), (b, s, d)
    return _sc_call(x, pos_emb[:s])
